# Initial kernel scaffold; baseline (speedup 1.0000x reference)
import jax
import jax.numpy as jnp
from jax.experimental import pallas as pl


def kernel(X, edge_index, edge_weight, F_param):
    raise NotImplementedError("write your pallas kernel here")


# speedup vs baseline: 3.5326x; 3.5326x over previous
import functools

import jax
import jax.numpy as jnp
from jax import lax
from jax.experimental import pallas as pl
from jax.experimental.pallas import tpu as pltpu
from jax.experimental.pallas import tpu_sc as plsc

N = 10000
M = 128
GAMMA = 0.8
MAX_ITER = 50
THRESHOLD = 1e-3
EPS_F = 1e-12

NC = 2
NS = 16
L = 16
NW = NC * NS
CH = 128
N_PAD = 10240
RPT = N_PAD // NS


def _spmm_sc(z, src3, dst3, w3, nchunk):
    mesh = plsc.VectorSubcoreMesh(
        core_axis_name="c", subcore_axis_name="s", num_cores=NC, num_subcores=NS
    )

    @functools.partial(
        pl.kernel,
        out_type=jax.ShapeDtypeStruct((NC, N_PAD, M), jnp.float32),
        mesh=mesh,
        compiler_params=pltpu.CompilerParams(needs_layout_passes=False),
        scratch_types=[
            pltpu.VMEM((nchunk, CH), jnp.int32),
            pltpu.VMEM((nchunk, CH), jnp.int32),
            pltpu.VMEM((nchunk * CH,), jnp.float32),
            pltpu.VMEM((CH, M), jnp.float32),
            pltpu.VMEM_SHARED((N_PAD, M), jnp.float32),
            pltpu.SemaphoreType.DMA,
        ],
    )
    def k(z_hbm, src_hbm, dst_hbm, w_hbm, p_hbm, src_v, dst_v, w_v, rows_v, acc, sem):
        c = lax.axis_index("c")
        s = lax.axis_index("s")
        wid = c * NS + s

        pltpu.sync_copy(src_hbm.at[wid], src_v)
        pltpu.sync_copy(dst_hbm.at[wid], dst_v)
        pltpu.sync_copy(w_hbm.at[wid], w_v)

        zero = jnp.zeros((L,), jnp.float32)

        def zrow(r, carry):
            for q in range(M // L):
                rows_v[r, pl.ds(q * L, L)] = zero
            return carry

        lax.fori_loop(0, CH, zrow, 0)

        base = s * RPT
        for t in range(RPT // CH):
            pltpu.sync_copy(rows_v, acc.at[pl.ds(base + t * CH, CH)])
        plsc.subcore_barrier()

        def chunk_body(j, carry):
            pltpu.async_copy(z_hbm.at[src_v.at[j]], rows_v, sem).wait()
            fbase = j * CH
            for e in range(CH):
                wsp = plsc.load_gather(w_v, [jnp.full((L,), fbase + e, jnp.int32)])
                for q in range(M // L):
                    rows_v[e, pl.ds(q * L, L)] = rows_v[e, pl.ds(q * L, L)] * wsp
            pltpu.sync_copy(rows_v, acc.at[dst_v.at[j]], add=True)
            return carry

        lax.fori_loop(0, nchunk, chunk_body, 0)

        plsc.subcore_barrier()
        pltpu.sync_copy(acc.at[pl.ds(base, RPT)], p_hbm.at[c, pl.ds(base, RPT)])

    return k(z, src3, dst3, w3)


def _g_of_f(F):
    def body(f_ref, g_ref):
        ff = lax.dot_general(
            f_ref[...], f_ref[...], (((0,), (0,)), ((), ())),
            preferred_element_type=jnp.float32,
        )
        n = jnp.sqrt(jnp.sum(ff * ff))
        g_ref[...] = ff / (n + EPS_F)

    return pl.pallas_call(
        body, out_shape=jax.ShapeDtypeStruct((M, M), jnp.float32)
    )(F)


_BR = 400


def _update_tc(P, X, Z, G):
    def body(p_ref, x_ref, z_ref, g_ref, zn_ref, d2_ref, n2_ref):
        i = pl.program_id(0)
        sp = p_ref[0] + p_ref[1]
        zn = GAMMA * jnp.dot(
            sp, g_ref[...], preferred_element_type=jnp.float32
        ) + x_ref[...]
        zn_ref[...] = zn
        d = zn - z_ref[...]
        pd = jnp.sum(d * d)
        pn = jnp.sum(zn * zn)

        @pl.when(i == 0)
        def _():
            d2_ref[0, 0] = pd
            n2_ref[0, 0] = pn

        @pl.when(i != 0)
        def _():
            d2_ref[0, 0] += pd
            n2_ref[0, 0] += pn

    return pl.pallas_call(
        body,
        grid=(N // _BR,),
        in_specs=[
            pl.BlockSpec((NC, _BR, M), lambda i: (0, i, 0)),
            pl.BlockSpec((_BR, M), lambda i: (i, 0)),
            pl.BlockSpec((_BR, M), lambda i: (i, 0)),
            pl.BlockSpec((M, M), lambda i: (0, 0)),
        ],
        out_specs=[
            pl.BlockSpec((_BR, M), lambda i: (i, 0)),
            pl.BlockSpec(memory_space=pltpu.SMEM),
            pl.BlockSpec(memory_space=pltpu.SMEM),
        ],
        out_shape=[
            jax.ShapeDtypeStruct((N, M), jnp.float32),
            jax.ShapeDtypeStruct((1, 1), jnp.float32),
            jax.ShapeDtypeStruct((1, 1), jnp.float32),
        ],
    )(P, X, Z, G)


def kernel(X, edge_index, edge_weight, F_param):
    E = edge_weight.shape[0]
    src = edge_index[0].astype(jnp.int32)
    dst = edge_index[1].astype(jnp.int32)
    w = edge_weight.astype(jnp.float32)

    epw = -(-E // NW)
    nchunk = -(-epw // CH)
    e_pad = NW * nchunk * CH
    pad = e_pad - E
    src3 = jnp.reshape(
        jnp.concatenate([src, jnp.zeros((pad,), jnp.int32)]), (NW, nchunk, CH)
    )
    dst3 = jnp.reshape(
        jnp.concatenate([dst, jnp.zeros((pad,), jnp.int32)]), (NW, nchunk, CH)
    )
    w3 = jnp.reshape(
        jnp.concatenate([w, jnp.zeros((pad,), jnp.float32)]), (NW, nchunk * CH)
    )

    G = _g_of_f(F_param)
    Z0 = jnp.zeros_like(X)

    def cond(st):
        _, i, done = st
        return jnp.logical_and(i < MAX_ITER, jnp.logical_not(done))

    def body(st):
        Z, i, _ = st
        P = _spmm_sc(Z, src3, dst3, w3, nchunk)
        Zn, d2, n2 = _update_tc(P, X, Z, G)
        diff = jnp.sqrt(d2[0, 0]) / (jnp.sqrt(n2[0, 0]) + 1e-9)
        return (Zn, i + 1, diff < THRESHOLD)

    Z, _, _ = lax.while_loop(
        cond, body, (Z0, jnp.asarray(0, jnp.int32), jnp.asarray(False))
    )
    return Z


# speedup vs baseline: 5.2492x; 1.4859x over previous
import functools

import jax
import jax.numpy as jnp
from jax import lax
from jax.experimental import pallas as pl
from jax.experimental.pallas import tpu as pltpu
from jax.experimental.pallas import tpu_sc as plsc

N = 10000
M = 128
GAMMA = 0.8
MAX_ITER = 50
THRESHOLD = 1e-3
EPS_F = 1e-12

NC = 2
NS = 16
L = 16
NW = NC * NS
CH = 128
N_PAD = 10240
RPT = N_PAD // NS


def _spmm_sc(z, src3, dst3, w3, nchunk):
    mesh = plsc.VectorSubcoreMesh(
        core_axis_name="c", subcore_axis_name="s", num_cores=NC, num_subcores=NS
    )

    @functools.partial(
        pl.kernel,
        out_type=jax.ShapeDtypeStruct((NC, N_PAD, M), jnp.float32),
        mesh=mesh,
        compiler_params=pltpu.CompilerParams(needs_layout_passes=False),
        scratch_types=[
            pltpu.VMEM((nchunk, CH), jnp.int32),
            pltpu.VMEM((nchunk, CH), jnp.int32),
            pltpu.VMEM((nchunk * CH,), jnp.float32),
            pltpu.VMEM((CH, M), jnp.float32),
            pltpu.VMEM_SHARED((N_PAD, M), jnp.float32),
            pltpu.SemaphoreType.DMA,
        ],
    )
    def k(z_hbm, src_hbm, dst_hbm, w_hbm, p_hbm, src_v, dst_v, w_v, rows_v, acc, sem):
        c = lax.axis_index("c")
        s = lax.axis_index("s")
        wid = c * NS + s

        pltpu.sync_copy(src_hbm.at[wid], src_v)
        pltpu.sync_copy(dst_hbm.at[wid], dst_v)
        pltpu.sync_copy(w_hbm.at[wid], w_v)

        zero = jnp.zeros((L,), jnp.float32)

        def zrow(r, carry):
            for q in range(M // L):
                rows_v[r, pl.ds(q * L, L)] = zero
            return carry

        lax.fori_loop(0, CH, zrow, 0)

        base = s * RPT
        for t in range(RPT // CH):
            pltpu.sync_copy(rows_v, acc.at[pl.ds(base + t * CH, CH)])
        plsc.subcore_barrier()

        def chunk_body(j, carry):
            pltpu.async_copy(z_hbm.at[src_v.at[j]], rows_v, sem).wait()
            fbase = j * CH
            for e in range(CH):
                wsp = plsc.load_gather(w_v, [jnp.full((L,), fbase + e, jnp.int32)])
                for q in range(M // L):
                    rows_v[e, pl.ds(q * L, L)] = rows_v[e, pl.ds(q * L, L)] * wsp
            pltpu.sync_copy(rows_v, acc.at[dst_v.at[j]], add=True)
            return carry

        lax.fori_loop(0, nchunk, chunk_body, 0)

        plsc.subcore_barrier()
        pltpu.sync_copy(acc.at[pl.ds(base, RPT)], p_hbm.at[c, pl.ds(base, RPT)])

    return k(z, src3, dst3, w3)


def _g_of_f(F):
    def body(f_ref, g_ref):
        ff = lax.dot_general(
            f_ref[...], f_ref[...], (((0,), (0,)), ((), ())),
            preferred_element_type=jnp.float32,
        )
        n = jnp.sqrt(jnp.sum(ff * ff))
        g_ref[...] = ff / (n + EPS_F)

    return pl.pallas_call(
        body, out_shape=jax.ShapeDtypeStruct((M, M), jnp.float32)
    )(F)


_BR = 400


def _update_tc(P, X, Z, G):
    def body(p_ref, x_ref, z_ref, g_ref, zn_ref, d2_ref, n2_ref):
        i = pl.program_id(0)
        sp = p_ref[0] + p_ref[1]
        zn = GAMMA * jnp.dot(
            sp, g_ref[...], preferred_element_type=jnp.float32
        ) + x_ref[...]
        zn_ref[...] = zn
        d = zn - z_ref[...]
        pd = jnp.sum(d * d)
        pn = jnp.sum(zn * zn)

        @pl.when(i == 0)
        def _():
            d2_ref[0, 0] = pd
            n2_ref[0, 0] = pn

        @pl.when(i != 0)
        def _():
            d2_ref[0, 0] += pd
            n2_ref[0, 0] += pn

    return pl.pallas_call(
        body,
        grid=(N // _BR,),
        in_specs=[
            pl.BlockSpec((NC, _BR, M), lambda i: (0, i, 0)),
            pl.BlockSpec((_BR, M), lambda i: (i, 0)),
            pl.BlockSpec((_BR, M), lambda i: (i, 0)),
            pl.BlockSpec((M, M), lambda i: (0, 0)),
        ],
        out_specs=[
            pl.BlockSpec((_BR, M), lambda i: (i, 0)),
            pl.BlockSpec(memory_space=pltpu.SMEM),
            pl.BlockSpec(memory_space=pltpu.SMEM),
        ],
        out_shape=[
            jax.ShapeDtypeStruct((N, M), jnp.float32),
            jax.ShapeDtypeStruct((1, 1), jnp.float32),
            jax.ShapeDtypeStruct((1, 1), jnp.float32),
        ],
    )(P, X, Z, G)


def kernel(X, edge_index, edge_weight, F_param):
    E = edge_weight.shape[0]
    src = edge_index[0].astype(jnp.int32)
    dst = edge_index[1].astype(jnp.int32)
    w = edge_weight.astype(jnp.float32)

    epw = -(-E // NW)
    nchunk = -(-epw // CH)
    e_pad = NW * nchunk * CH
    pad = e_pad - E
    src3 = jnp.reshape(
        jnp.concatenate([src, jnp.zeros((pad,), jnp.int32)]), (NW, nchunk, CH)
    )
    dst3 = jnp.reshape(
        jnp.concatenate([dst, jnp.zeros((pad,), jnp.int32)]), (NW, nchunk, CH)
    )
    w3 = jnp.reshape(
        jnp.concatenate([w, jnp.zeros((pad,), jnp.float32)]), (NW, nchunk * CH)
    )

    G = _g_of_f(F_param)

    def cond(st):
        _, i, done = st
        return jnp.logical_and(i < MAX_ITER, jnp.logical_not(done))

    def body(st):
        Z, i, _ = st
        P = _spmm_sc(Z, src3, dst3, w3, nchunk)
        Zn, d2, n2 = _update_tc(P, X, Z, G)
        diff = jnp.sqrt(d2[0, 0]) / (jnp.sqrt(n2[0, 0]) + 1e-9)
        return (Zn, i + 1, diff < THRESHOLD)

    Z, _, _ = lax.while_loop(
        cond, body, (X, jnp.asarray(1, jnp.int32), jnp.asarray(False))
    )
    return Z
